# baseline (speedup 1.0000x reference)
import jax
import jax.numpy as jnp
from jax.experimental import pallas as pl


def _onehot_kernel(lab_ref, out_ref):
    col = jax.lax.broadcasted_iota(jnp.int32, out_ref.shape, 1)
    out_ref[...] = (lab_ref[...] == col).astype(jnp.float32)


def kernel(input_ids, attention_mask, labels):
    batch = input_ids.shape[0]
    lab = labels.reshape(labels.shape[0], 1)
    return pl.pallas_call(
        _onehot_kernel,
        out_shape=jax.ShapeDtypeStruct((batch, 2), jnp.float32),
        grid=(1,),
        in_specs=[pl.BlockSpec((batch, 1), lambda i: (0, 0))],
        out_specs=pl.BlockSpec((batch, 2), lambda i: (0, 0)),
    )(lab)
